# baseline (speedup 1.0000x reference)
import functools

import jax
import jax.numpy as jnp
from jax import lax
from jax.experimental import pallas as pl
from jax.experimental.pallas import tpu as pltpu
from jax.experimental.pallas import tpu_sc as plsc

B = 16384
D = 64
L = 16

_info = plsc.get_sparse_core_info()
NC, NS = _info.num_cores, _info.num_subcores
NW = NC * NS
BPW = B // NW

_mesh = plsc.VectorSubcoreMesh(core_axis_name="c", subcore_axis_name="s")


@functools.partial(
    pl.kernel,
    mesh=_mesh,
    out_type=(
        jax.ShapeDtypeStruct((B,), jnp.float32),
        jax.ShapeDtypeStruct((B,), jnp.float32),
    ),
    compiler_params=pltpu.CompilerParams(use_tc_tiling_on_sc=False),
    scratch_types=[
        pltpu.VMEM((BPW,), jnp.int32),
        pltpu.VMEM((BPW,), jnp.int32),
        pltpu.VMEM((BPW,), jnp.int32),
        pltpu.VMEM((BPW, D), jnp.float32),
        pltpu.VMEM((BPW, D), jnp.float32),
        pltpu.VMEM((BPW, D), jnp.float32),
        pltpu.VMEM((BPW,), jnp.float32),
        pltpu.VMEM((BPW,), jnp.float32),
        pltpu.SemaphoreType.DMA,
    ],
)
def _bpr_kernel(user_hbm, item_i_hbm, item_j_hbm, uw_hbm, iw_hbm,
                out_i_hbm, out_j_hbm,
                idx_u, idx_i, idx_j, rows_u, rows_i, rows_j,
                out_i_v, out_j_v, sem):
    wid = lax.axis_index("s") * NC + lax.axis_index("c")
    base = wid * BPW

    pltpu.sync_copy(user_hbm.at[pl.ds(base, BPW)], idx_u)
    pltpu.sync_copy(item_i_hbm.at[pl.ds(base, BPW)], idx_i)
    pltpu.sync_copy(item_j_hbm.at[pl.ds(base, BPW)], idx_j)

    cu = pltpu.make_async_copy(uw_hbm.at[idx_u], rows_u, sem)
    ci = pltpu.make_async_copy(iw_hbm.at[idx_i], rows_i, sem)
    cj = pltpu.make_async_copy(iw_hbm.at[idx_j], rows_j, sem)
    cu.start()
    ci.start()
    cj.start()
    cu.wait()
    ci.wait()
    cj.wait()

    lane = lax.iota(jnp.int32, L)
    perms = [lane ^ s for s in (1, 2, 4, 8)]

    def lanesum(v):
        for p in perms:
            v = v + v.at[p].get(mode="promise_in_bounds")
        return v

    def group(g, _):
        acc_i = jnp.zeros((L,), jnp.float32)
        acc_j = jnp.zeros((L,), jnp.float32)
        for r in range(L):
            b = g * L + r
            u0 = rows_u[b, pl.ds(0, L)]
            u1 = rows_u[b, pl.ds(L, L)]
            u2 = rows_u[b, pl.ds(2 * L, L)]
            u3 = rows_u[b, pl.ds(3 * L, L)]
            i0 = rows_i[b, pl.ds(0, L)]
            i1 = rows_i[b, pl.ds(L, L)]
            i2 = rows_i[b, pl.ds(2 * L, L)]
            i3 = rows_i[b, pl.ds(3 * L, L)]
            j0 = rows_j[b, pl.ds(0, L)]
            j1 = rows_j[b, pl.ds(L, L)]
            j2 = rows_j[b, pl.ds(2 * L, L)]
            j3 = rows_j[b, pl.ds(3 * L, L)]
            pi = (u0 * i0 + u1 * i1) + (u2 * i2 + u3 * i3)
            pj = (u0 * j0 + u1 * j1) + (u2 * j2 + u3 * j3)
            sel = lane == r
            acc_i = jnp.where(sel, lanesum(pi), acc_i)
            acc_j = jnp.where(sel, lanesum(pj), acc_j)
        out_i_v[pl.ds(g * L, L)] = acc_i
        out_j_v[pl.ds(g * L, L)] = acc_j
        return 0

    lax.fori_loop(0, BPW // L, group, 0)

    pltpu.sync_copy(out_i_v, out_i_hbm.at[pl.ds(base, BPW)])
    pltpu.sync_copy(out_j_v, out_j_hbm.at[pl.ds(base, BPW)])


def kernel(user, item_i, item_j, user_emb_weight, item_emb_weight):
    return _bpr_kernel(user, item_i, item_j, user_emb_weight, item_emb_weight)


# speedup vs baseline: 1.6389x; 1.6389x over previous
import functools

import jax
import jax.numpy as jnp
from jax import lax
from jax.experimental import pallas as pl
from jax.experimental.pallas import tpu as pltpu
from jax.experimental.pallas import tpu_sc as plsc

B = 16384
D = 64
L = 16
V = 1000000


_TBLK = 4096
_TGRID = (V + _TBLK - 1) // _TBLK
_VPAD = _TGRID * _TBLK


def _transpose_body(x_ref, out_ref):
    x = x_ref[...]
    row = lax.broadcasted_iota(jnp.int32, (D, D), 0)
    col = lax.broadcasted_iota(jnp.int32, (D, D), 1)
    ident = (row == col).astype(jnp.float32)
    y = lax.dot_general(x, ident, (((0,), (0,)), ((), ())),
                        preferred_element_type=jnp.float32)
    out_ref[...] = jnp.concatenate(
        [y[: _TBLK // 2], y[_TBLK // 2 :]], axis=1)


_tc_detile = pl.pallas_call(
    _transpose_body,
    grid=(_TGRID,),
    in_specs=[pl.BlockSpec((D, _TBLK), lambda b: (0, b))],
    out_specs=pl.BlockSpec((_TBLK // 2, 2 * D), lambda b: (b, 0)),
    out_shape=jax.ShapeDtypeStruct((_VPAD // 2, 2 * D), jnp.float32),
)


_info = plsc.get_sparse_core_info()
NC, NS = _info.num_cores, _info.num_subcores
NW = NC * NS
BPW = B // NW

_mesh = plsc.VectorSubcoreMesh(core_axis_name="c", subcore_axis_name="s")


@functools.partial(
    pl.kernel,
    mesh=_mesh,
    out_type=(
        jax.ShapeDtypeStruct((B,), jnp.float32),
        jax.ShapeDtypeStruct((B,), jnp.float32),
    ),
    compiler_params=pltpu.CompilerParams(use_tc_tiling_on_sc=False),
    scratch_types=[
        pltpu.VMEM((BPW,), jnp.int32),
        pltpu.VMEM((BPW,), jnp.int32),
        pltpu.VMEM((BPW,), jnp.int32),
        pltpu.VMEM((BPW,), jnp.int32),
        pltpu.VMEM((BPW,), jnp.int32),
        pltpu.VMEM((BPW,), jnp.int32),
        pltpu.VMEM((BPW, D), jnp.float32),
        pltpu.VMEM((BPW, D), jnp.float32),
        pltpu.VMEM((BPW, D), jnp.float32),
        pltpu.VMEM((BPW,), jnp.float32),
        pltpu.VMEM((BPW,), jnp.float32),
        pltpu.SemaphoreType.DMA,
    ],
)
def _bpr_kernel(user_hbm, item_i_hbm, item_j_hbm, uw_hbm, iw_hbm,
                out_i_hbm, out_j_hbm,
                idx_u, idx_i, idx_j, idxq_u, idxq_i, idxq_j,
                rows_u, rows_i, rows_j,
                out_i_v, out_j_v, sem):
    wid = lax.axis_index("s") * NC + lax.axis_index("c")
    base = wid * BPW

    pltpu.sync_copy(user_hbm.at[pl.ds(base, BPW)], idx_u)
    pltpu.sync_copy(item_i_hbm.at[pl.ds(base, BPW)], idx_i)
    pltpu.sync_copy(item_j_hbm.at[pl.ds(base, BPW)], idx_j)

    def remap(g, _):
        s = pl.ds(g * L, L)
        for src, dst in ((idx_u, idxq_u), (idx_i, idxq_i), (idx_j, idxq_j)):
            v = src[s]
            dst[s] = ((v >> 12) << 12) + ((v & 2047) << 1) + ((v >> 11) & 1)
        return 0

    lax.fori_loop(0, BPW // L, remap, 0)

    cu = pltpu.make_async_copy(uw_hbm.at[idxq_u], rows_u, sem)
    ci = pltpu.make_async_copy(iw_hbm.at[idxq_i], rows_i, sem)
    cj = pltpu.make_async_copy(iw_hbm.at[idxq_j], rows_j, sem)
    cu.start()
    ci.start()
    cj.start()
    cu.wait()
    ci.wait()
    cj.wait()

    lane = lax.iota(jnp.int32, L)
    perms = [lane ^ s for s in (1, 2, 4, 8)]

    def lanesum(v):
        for p in perms:
            v = v + v.at[p].get(mode="promise_in_bounds")
        return v

    def group(g, _):
        acc_i = jnp.zeros((L,), jnp.float32)
        acc_j = jnp.zeros((L,), jnp.float32)
        for r in range(L):
            b = g * L + r
            u0 = rows_u[b, pl.ds(0, L)]
            u1 = rows_u[b, pl.ds(L, L)]
            u2 = rows_u[b, pl.ds(2 * L, L)]
            u3 = rows_u[b, pl.ds(3 * L, L)]
            i0 = rows_i[b, pl.ds(0, L)]
            i1 = rows_i[b, pl.ds(L, L)]
            i2 = rows_i[b, pl.ds(2 * L, L)]
            i3 = rows_i[b, pl.ds(3 * L, L)]
            j0 = rows_j[b, pl.ds(0, L)]
            j1 = rows_j[b, pl.ds(L, L)]
            j2 = rows_j[b, pl.ds(2 * L, L)]
            j3 = rows_j[b, pl.ds(3 * L, L)]
            pi = (u0 * i0 + u1 * i1) + (u2 * i2 + u3 * i3)
            pj = (u0 * j0 + u1 * j1) + (u2 * j2 + u3 * j3)
            sel = lane == r
            acc_i = jnp.where(sel, lanesum(pi), acc_i)
            acc_j = jnp.where(sel, lanesum(pj), acc_j)
        out_i_v[pl.ds(g * L, L)] = acc_i
        out_j_v[pl.ds(g * L, L)] = acc_j
        return 0

    lax.fori_loop(0, BPW // L, group, 0)

    pltpu.sync_copy(out_i_v, out_i_hbm.at[pl.ds(base, BPW)])
    pltpu.sync_copy(out_j_v, out_j_hbm.at[pl.ds(base, BPW)])


def kernel(user, item_i, item_j, user_emb_weight, item_emb_weight):
    uw_lin = _tc_detile(user_emb_weight.T).reshape(_VPAD, D)
    iw_lin = _tc_detile(item_emb_weight.T).reshape(_VPAD, D)
    return _bpr_kernel(user, item_i, item_j, uw_lin, iw_lin)


# speedup vs baseline: 1.6424x; 1.0022x over previous
import functools

import jax
import jax.numpy as jnp
from jax import lax
from jax.experimental import pallas as pl
from jax.experimental.pallas import tpu as pltpu
from jax.experimental.pallas import tpu_sc as plsc

B = 16384
D = 64
L = 16
V = 1000000


_TBLK = 4096
_TGRID = (V + _TBLK - 1) // _TBLK
_VPAD = _TGRID * _TBLK


def _transpose_body(x_ref, out_ref):
    x = x_ref[...]
    y = jnp.swapaxes(x, 0, 1)
    out_ref[...] = jnp.concatenate(
        [y[: _TBLK // 2], y[_TBLK // 2 :]], axis=1)


_tc_detile = pl.pallas_call(
    _transpose_body,
    grid=(_TGRID,),
    in_specs=[pl.BlockSpec((D, _TBLK), lambda b: (0, b))],
    out_specs=pl.BlockSpec((_TBLK // 2, 2 * D), lambda b: (b, 0)),
    out_shape=jax.ShapeDtypeStruct((_VPAD // 2, 2 * D), jnp.float32),
)


_info = plsc.get_sparse_core_info()
NC, NS = _info.num_cores, _info.num_subcores
NW = NC * NS
BPW = B // NW

_mesh = plsc.VectorSubcoreMesh(core_axis_name="c", subcore_axis_name="s")


@functools.partial(
    pl.kernel,
    mesh=_mesh,
    out_type=(
        jax.ShapeDtypeStruct((B,), jnp.float32),
        jax.ShapeDtypeStruct((B,), jnp.float32),
    ),
    compiler_params=pltpu.CompilerParams(use_tc_tiling_on_sc=False),
    scratch_types=[
        pltpu.VMEM((BPW,), jnp.int32),
        pltpu.VMEM((BPW,), jnp.int32),
        pltpu.VMEM((BPW,), jnp.int32),
        pltpu.VMEM((BPW,), jnp.int32),
        pltpu.VMEM((BPW,), jnp.int32),
        pltpu.VMEM((BPW,), jnp.int32),
        pltpu.VMEM((BPW, D), jnp.float32),
        pltpu.VMEM((BPW, D), jnp.float32),
        pltpu.VMEM((BPW, D), jnp.float32),
        pltpu.VMEM((BPW,), jnp.float32),
        pltpu.VMEM((BPW,), jnp.float32),
        pltpu.SemaphoreType.DMA,
    ],
)
def _bpr_kernel(user_hbm, item_i_hbm, item_j_hbm, uw_hbm, iw_hbm,
                out_i_hbm, out_j_hbm,
                idx_u, idx_i, idx_j, idxq_u, idxq_i, idxq_j,
                rows_u, rows_i, rows_j,
                out_i_v, out_j_v, sem):
    wid = lax.axis_index("s") * NC + lax.axis_index("c")
    base = wid * BPW

    pltpu.sync_copy(user_hbm.at[pl.ds(base, BPW)], idx_u)
    pltpu.sync_copy(item_i_hbm.at[pl.ds(base, BPW)], idx_i)
    pltpu.sync_copy(item_j_hbm.at[pl.ds(base, BPW)], idx_j)

    def remap(g, _):
        s = pl.ds(g * L, L)
        for src, dst in ((idx_u, idxq_u), (idx_i, idxq_i), (idx_j, idxq_j)):
            v = src[s]
            dst[s] = ((v >> 12) << 12) + ((v & 2047) << 1) + ((v >> 11) & 1)
        return 0

    lax.fori_loop(0, BPW // L, remap, 0)

    cu = pltpu.make_async_copy(uw_hbm.at[idxq_u], rows_u, sem)
    ci = pltpu.make_async_copy(iw_hbm.at[idxq_i], rows_i, sem)
    cj = pltpu.make_async_copy(iw_hbm.at[idxq_j], rows_j, sem)
    cu.start()
    ci.start()
    cj.start()
    cu.wait()
    ci.wait()
    cj.wait()

    lane = lax.iota(jnp.int32, L)
    perms = [lane ^ s for s in (1, 2, 4, 8)]

    def lanesum(v):
        for p in perms:
            v = v + v.at[p].get(mode="promise_in_bounds")
        return v

    def group(g, _):
        acc_i = jnp.zeros((L,), jnp.float32)
        acc_j = jnp.zeros((L,), jnp.float32)
        for r in range(L):
            b = g * L + r
            u0 = rows_u[b, pl.ds(0, L)]
            u1 = rows_u[b, pl.ds(L, L)]
            u2 = rows_u[b, pl.ds(2 * L, L)]
            u3 = rows_u[b, pl.ds(3 * L, L)]
            i0 = rows_i[b, pl.ds(0, L)]
            i1 = rows_i[b, pl.ds(L, L)]
            i2 = rows_i[b, pl.ds(2 * L, L)]
            i3 = rows_i[b, pl.ds(3 * L, L)]
            j0 = rows_j[b, pl.ds(0, L)]
            j1 = rows_j[b, pl.ds(L, L)]
            j2 = rows_j[b, pl.ds(2 * L, L)]
            j3 = rows_j[b, pl.ds(3 * L, L)]
            pi = (u0 * i0 + u1 * i1) + (u2 * i2 + u3 * i3)
            pj = (u0 * j0 + u1 * j1) + (u2 * j2 + u3 * j3)
            sel = lane == r
            acc_i = jnp.where(sel, lanesum(pi), acc_i)
            acc_j = jnp.where(sel, lanesum(pj), acc_j)
        out_i_v[pl.ds(g * L, L)] = acc_i
        out_j_v[pl.ds(g * L, L)] = acc_j
        return 0

    lax.fori_loop(0, BPW // L, group, 0)

    pltpu.sync_copy(out_i_v, out_i_hbm.at[pl.ds(base, BPW)])
    pltpu.sync_copy(out_j_v, out_j_hbm.at[pl.ds(base, BPW)])


def kernel(user, item_i, item_j, user_emb_weight, item_emb_weight):
    uw_lin = _tc_detile(user_emb_weight.T).reshape(_VPAD, D)
    iw_lin = _tc_detile(item_emb_weight.T).reshape(_VPAD, D)
    return _bpr_kernel(user, item_i, item_j, uw_lin, iw_lin)


# speedup vs baseline: 1.9768x; 1.2036x over previous
import functools

import jax
import jax.numpy as jnp
from jax import lax
from jax.experimental import pallas as pl
from jax.experimental.pallas import tpu as pltpu
from jax.experimental.pallas import tpu_sc as plsc

B = 16384
D = 64
L = 16
V = 1000000


_TBLK = 32768
_TSH = _TBLK.bit_length() - 1
_TGRID = (V + _TBLK - 1) // _TBLK
_VPAD = _TGRID * _TBLK


def _transpose_body(x_ref, out_ref):
    x = x_ref[...]
    y = jnp.swapaxes(x, 0, 1)
    out_ref[...] = jnp.concatenate(
        [y[: _TBLK // 2], y[_TBLK // 2 :]], axis=1)


_tc_detile = pl.pallas_call(
    _transpose_body,
    grid=(_TGRID,),
    in_specs=[pl.BlockSpec((D, _TBLK), lambda b: (0, b))],
    out_specs=pl.BlockSpec((_TBLK // 2, 2 * D), lambda b: (b, 0)),
    out_shape=jax.ShapeDtypeStruct((_VPAD // 2, 2 * D), jnp.float32),
)


_info = plsc.get_sparse_core_info()
NC, NS = _info.num_cores, _info.num_subcores
NW = NC * NS
BPW = B // NW

_mesh = plsc.VectorSubcoreMesh(core_axis_name="c", subcore_axis_name="s")


_NWIN = (V + 127) // 128
_NFULL = _NWIN - 1
_UVP = _NWIN * 128
_WITER = (_NFULL + NW - 1) // NW


def _win_transpose(win, rows, lane, nv):
    def vblock(vb, _):
        for k in range(L):
            v_idx = (vb * L + k) ^ lane
            for c in range(D // L):
                d_idx = c * L + lane
                g = plsc.load_gather(win, [d_idx, v_idx])
                plsc.store_scatter(rows, [(v_idx << 6) + d_idx], g)
        return 0

    lax.fori_loop(0, nv // L, vblock, 0)


@functools.partial(
    pl.kernel,
    mesh=_mesh,
    out_type=jax.ShapeDtypeStruct((_UVP * D,), jnp.float32),
    compiler_params=pltpu.CompilerParams(needs_layout_passes=False),
    scratch_types=[
        pltpu.VMEM((2 * D, 128), jnp.float32),
        pltpu.VMEM((D, D), jnp.float32),
        pltpu.VMEM((2 * 128 * D,), jnp.float32),
        pltpu.SemaphoreType.DMA,
        pltpu.SemaphoreType.DMA,
    ],
)
def _sc_uconv(uwt_hbm, out_hbm, win2, wintail, rows2, sem_in, sem_out):
    wid = lax.axis_index("s") * NC + lax.axis_index("c")
    lane = lax.iota(jnp.int32, L)

    def fire(i, buf):
        w = i * NW + wid

        @pl.when(w < _NFULL)
        def _():
            pltpu.make_async_copy(
                uwt_hbm.at[:, pl.ds(pl.multiple_of(w * 128, 128), 128)],
                win2.at[pl.ds(buf * D, D), :], sem_in).start()

    def process(i, buf):
        w = i * NW + wid

        @pl.when(w < _NFULL)
        def _():
            @pl.when(i >= 2)
            def _():
                pltpu.make_async_copy(
                    rows2.at[pl.ds(buf * 8192, 8192)], out_hbm.at[pl.ds(0, 8192)],
                    sem_out).wait()

            pltpu.make_async_copy(
                uwt_hbm.at[:, pl.ds(0, 128)], win2.at[pl.ds(buf * D, D), :], sem_in).wait()
            _win_transpose(win2.at[pl.ds(buf * D, D), :], rows2.at[pl.ds(buf * 8192, 8192)], lane, 128)
            pltpu.make_async_copy(
                rows2.at[pl.ds(buf * 8192, 8192)],
                out_hbm.at[pl.ds(pl.multiple_of(w * 8192, 8), 8192)],
                sem_out).start()

    fire(0, 0)

    def pair(i2, _):
        i = i2 * 2
        fire(i + 1, 1)
        process(i, 0)
        fire(i + 2, 0)
        process(i + 1, 1)
        return 0

    lax.fori_loop(0, (_WITER + 1) // 2, pair, 0)

    pltpu.make_async_copy(rows2.at[pl.ds(0, 8192)], out_hbm.at[pl.ds(0, 8192)],
                          sem_out).wait()
    pltpu.make_async_copy(rows2.at[pl.ds(8192, 8192)], out_hbm.at[pl.ds(0, 8192)],
                          sem_out).wait()

    @pl.when(wid == 0)
    def _():
        pltpu.sync_copy(uwt_hbm.at[:, pl.ds(_NFULL * 128, 64)], wintail)

        def drow(d, _):
            for c in range(4):
                g = wintail[d, pl.ds(c * L, L)]
                plsc.store_scatter(
                    rows2.at[pl.ds(0, 8192)], [(c * L + lane) * D + d], g)
            return 0

        lax.fori_loop(0, D, drow, 0)
        pltpu.sync_copy(rows2.at[pl.ds(0, 8192)],
                        out_hbm.at[pl.ds(_NFULL * 8192, 8192)])


@functools.partial(
    pl.kernel,
    mesh=_mesh,
    out_type=(
        jax.ShapeDtypeStruct((B,), jnp.float32),
        jax.ShapeDtypeStruct((B,), jnp.float32),
    ),
    compiler_params=pltpu.CompilerParams(use_tc_tiling_on_sc=False),
    scratch_types=[
        pltpu.VMEM((BPW,), jnp.int32),
        pltpu.VMEM((BPW,), jnp.int32),
        pltpu.VMEM((BPW,), jnp.int32),
        pltpu.VMEM((BPW,), jnp.int32),
        pltpu.VMEM((BPW,), jnp.int32),
        pltpu.VMEM((BPW, D), jnp.float32),
        pltpu.VMEM((BPW, D), jnp.float32),
        pltpu.VMEM((BPW, D), jnp.float32),
        pltpu.VMEM((BPW,), jnp.float32),
        pltpu.VMEM((BPW,), jnp.float32),
        pltpu.SemaphoreType.DMA,
    ],
)
def _bpr_kernel(user_hbm, item_i_hbm, item_j_hbm, uw_hbm, iw_hbm,
                out_i_hbm, out_j_hbm,
                idx_u, idx_i, idx_j, idxq_i, idxq_j, rows_u, rows_i, rows_j,
                out_i_v, out_j_v, sem):
    wid = lax.axis_index("s") * NC + lax.axis_index("c")
    base = wid * BPW

    pltpu.sync_copy(user_hbm.at[pl.ds(base, BPW)], idx_u)
    pltpu.sync_copy(item_i_hbm.at[pl.ds(base, BPW)], idx_i)
    pltpu.sync_copy(item_j_hbm.at[pl.ds(base, BPW)], idx_j)

    cu = pltpu.make_async_copy(uw_hbm.at[idx_u], rows_u, sem)
    cu.start()

    def remap(g, _):
        s = pl.ds(g * L, L)
        for src, dst in ((idx_i, idxq_i), (idx_j, idxq_j)):
            v = src[s]
            dst[s] = (((v >> _TSH) << _TSH)
                      + ((v & (_TBLK // 2 - 1)) << 1)
                      + ((v >> (_TSH - 1)) & 1))
        return 0

    lax.fori_loop(0, BPW // L, remap, 0)

    ci = pltpu.make_async_copy(iw_hbm.at[idxq_i], rows_i, sem)
    cj = pltpu.make_async_copy(iw_hbm.at[idxq_j], rows_j, sem)
    ci.start()
    cj.start()
    cu.wait()
    ci.wait()
    cj.wait()

    lane = lax.iota(jnp.int32, L)
    perms = [lane ^ s for s in (1, 2, 4, 8)]

    def lanesum(v):
        for p in perms:
            v = v + v.at[p].get(mode="promise_in_bounds")
        return v

    def group(g, _):
        acc_i = jnp.zeros((L,), jnp.float32)
        acc_j = jnp.zeros((L,), jnp.float32)
        for r in range(L):
            b = g * L + r
            u0 = rows_u[b, pl.ds(0, L)]
            u1 = rows_u[b, pl.ds(L, L)]
            u2 = rows_u[b, pl.ds(2 * L, L)]
            u3 = rows_u[b, pl.ds(3 * L, L)]
            i0 = rows_i[b, pl.ds(0, L)]
            i1 = rows_i[b, pl.ds(L, L)]
            i2 = rows_i[b, pl.ds(2 * L, L)]
            i3 = rows_i[b, pl.ds(3 * L, L)]
            j0 = rows_j[b, pl.ds(0, L)]
            j1 = rows_j[b, pl.ds(L, L)]
            j2 = rows_j[b, pl.ds(2 * L, L)]
            j3 = rows_j[b, pl.ds(3 * L, L)]
            pi = (u0 * i0 + u1 * i1) + (u2 * i2 + u3 * i3)
            pj = (u0 * j0 + u1 * j1) + (u2 * j2 + u3 * j3)
            sel = lane == r
            acc_i = jnp.where(sel, lanesum(pi), acc_i)
            acc_j = jnp.where(sel, lanesum(pj), acc_j)
        out_i_v[pl.ds(g * L, L)] = acc_i
        out_j_v[pl.ds(g * L, L)] = acc_j
        return 0

    lax.fori_loop(0, BPW // L, group, 0)

    pltpu.sync_copy(out_i_v, out_i_hbm.at[pl.ds(base, BPW)])
    pltpu.sync_copy(out_j_v, out_j_hbm.at[pl.ds(base, BPW)])


def kernel(user, item_i, item_j, user_emb_weight, item_emb_weight):
    uw_lin = _sc_uconv(user_emb_weight.T).reshape(_UVP, D)
    iw_lin = _tc_detile(item_emb_weight.T).reshape(_VPAD, D)
    return _bpr_kernel(user, item_i, item_j, uw_lin, iw_lin)


# speedup vs baseline: 2.4022x; 1.2152x over previous
import functools

import jax
import jax.numpy as jnp
from jax import lax
from jax.experimental import pallas as pl
from jax.experimental.pallas import tpu as pltpu
from jax.experimental.pallas import tpu_sc as plsc

B = 16384
D = 64
L = 16
V = 1000000


_TBLK = 32768
_TSH = _TBLK.bit_length() - 1
_TGRID = (V + _TBLK - 1) // _TBLK
_VPAD = _TGRID * _TBLK


def _transpose_body(x_ref, out_ref):
    x = x_ref[...]
    y = jnp.swapaxes(x, 0, 1)
    out_ref[...] = jnp.concatenate(
        [y[: _TBLK // 2], y[_TBLK // 2 :]], axis=1)


_tc_detile = pl.pallas_call(
    _transpose_body,
    grid=(_TGRID,),
    in_specs=[pl.BlockSpec((D, _TBLK), lambda b: (0, b))],
    out_specs=pl.BlockSpec((_TBLK // 2, 2 * D), lambda b: (b, 0)),
    out_shape=jax.ShapeDtypeStruct((_VPAD // 2, 2 * D), jnp.float32),
)


_info = plsc.get_sparse_core_info()
NC, NS = _info.num_cores, _info.num_subcores
NW = NC * NS
BPW = B // NW

_mesh = plsc.VectorSubcoreMesh(core_axis_name="c", subcore_axis_name="s")


_NWIN = (V + 127) // 128
_NFULL = _NWIN - 1
_UVP = _NWIN * 128
_WITER = (_NFULL + NW - 1) // NW


def _win_transpose(win, rows, lane, nv):
    @plsc.parallel_loop(0, nv // L, unroll=2)
    def vblock(vb):
        for k in range(L):
            v_idx = (vb * L + k) ^ lane
            for c in range(D // L):
                d_idx = c * L + lane
                g = plsc.load_gather(win, [d_idx, v_idx])
                plsc.store_scatter(rows, [(v_idx << 6) + d_idx], g)


@functools.partial(
    pl.kernel,
    mesh=_mesh,
    out_type=jax.ShapeDtypeStruct((_UVP * D,), jnp.float32),
    compiler_params=pltpu.CompilerParams(needs_layout_passes=False),
    scratch_types=[
        pltpu.VMEM((2 * D, 128), jnp.float32),
        pltpu.VMEM((D, D), jnp.float32),
        pltpu.VMEM((2 * 128 * D,), jnp.float32),
        pltpu.SemaphoreType.DMA,
        pltpu.SemaphoreType.DMA,
    ],
)
def _sc_uconv(uwt_hbm, out_hbm, win2, wintail, rows2, sem_in, sem_out):
    wid = lax.axis_index("s") * NC + lax.axis_index("c")
    lane = lax.iota(jnp.int32, L)

    def fire(i, buf):
        w = i * NW + wid

        @pl.when(w < _NFULL)
        def _():
            pltpu.make_async_copy(
                uwt_hbm.at[:, pl.ds(pl.multiple_of(w * 128, 128), 128)],
                win2.at[pl.ds(buf * D, D), :], sem_in).start()

    def process(i, buf):
        w = i * NW + wid

        @pl.when(w < _NFULL)
        def _():
            @pl.when(i >= 2)
            def _():
                pltpu.make_async_copy(
                    rows2.at[pl.ds(buf * 8192, 8192)], out_hbm.at[pl.ds(0, 8192)],
                    sem_out).wait()

            pltpu.make_async_copy(
                uwt_hbm.at[:, pl.ds(0, 128)], win2.at[pl.ds(buf * D, D), :], sem_in).wait()
            _win_transpose(win2.at[pl.ds(buf * D, D), :], rows2.at[pl.ds(buf * 8192, 8192)], lane, 128)
            pltpu.make_async_copy(
                rows2.at[pl.ds(buf * 8192, 8192)],
                out_hbm.at[pl.ds(pl.multiple_of(w * 8192, 8), 8192)],
                sem_out).start()

    fire(0, 0)

    def pair(i2, _):
        i = i2 * 2
        fire(i + 1, 1)
        process(i, 0)
        fire(i + 2, 0)
        process(i + 1, 1)
        return 0

    lax.fori_loop(0, (_WITER + 1) // 2, pair, 0)

    pltpu.make_async_copy(rows2.at[pl.ds(0, 8192)], out_hbm.at[pl.ds(0, 8192)],
                          sem_out).wait()
    pltpu.make_async_copy(rows2.at[pl.ds(8192, 8192)], out_hbm.at[pl.ds(0, 8192)],
                          sem_out).wait()

    @pl.when(wid == 0)
    def _():
        pltpu.sync_copy(uwt_hbm.at[:, pl.ds(_NFULL * 128, 64)], wintail)

        def drow(d, _):
            for c in range(4):
                g = wintail[d, pl.ds(c * L, L)]
                plsc.store_scatter(
                    rows2.at[pl.ds(0, 8192)], [(c * L + lane) * D + d], g)
            return 0

        lax.fori_loop(0, D, drow, 0)
        pltpu.sync_copy(rows2.at[pl.ds(0, 8192)],
                        out_hbm.at[pl.ds(_NFULL * 8192, 8192)])


@functools.partial(
    pl.kernel,
    mesh=_mesh,
    out_type=(
        jax.ShapeDtypeStruct((B,), jnp.float32),
        jax.ShapeDtypeStruct((B,), jnp.float32),
    ),
    compiler_params=pltpu.CompilerParams(use_tc_tiling_on_sc=False),
    scratch_types=[
        pltpu.VMEM((BPW,), jnp.int32),
        pltpu.VMEM((BPW,), jnp.int32),
        pltpu.VMEM((BPW,), jnp.int32),
        pltpu.VMEM((BPW,), jnp.int32),
        pltpu.VMEM((BPW,), jnp.int32),
        pltpu.VMEM((BPW, D), jnp.float32),
        pltpu.VMEM((BPW, D), jnp.float32),
        pltpu.VMEM((BPW, D), jnp.float32),
        pltpu.VMEM((BPW,), jnp.float32),
        pltpu.VMEM((BPW,), jnp.float32),
        pltpu.SemaphoreType.DMA,
    ],
)
def _bpr_kernel(user_hbm, item_i_hbm, item_j_hbm, uw_hbm, iw_hbm,
                out_i_hbm, out_j_hbm,
                idx_u, idx_i, idx_j, idxq_i, idxq_j, rows_u, rows_i, rows_j,
                out_i_v, out_j_v, sem):
    wid = lax.axis_index("s") * NC + lax.axis_index("c")
    base = wid * BPW

    pltpu.sync_copy(user_hbm.at[pl.ds(base, BPW)], idx_u)
    pltpu.sync_copy(item_i_hbm.at[pl.ds(base, BPW)], idx_i)
    pltpu.sync_copy(item_j_hbm.at[pl.ds(base, BPW)], idx_j)

    cu = pltpu.make_async_copy(uw_hbm.at[idx_u], rows_u, sem)
    cu.start()

    def remap(g, _):
        s = pl.ds(g * L, L)
        for src, dst in ((idx_i, idxq_i), (idx_j, idxq_j)):
            v = src[s]
            dst[s] = (((v >> _TSH) << _TSH)
                      + ((v & (_TBLK // 2 - 1)) << 1)
                      + ((v >> (_TSH - 1)) & 1))
        return 0

    lax.fori_loop(0, BPW // L, remap, 0)

    ci = pltpu.make_async_copy(iw_hbm.at[idxq_i], rows_i, sem)
    cj = pltpu.make_async_copy(iw_hbm.at[idxq_j], rows_j, sem)
    ci.start()
    cj.start()
    cu.wait()
    ci.wait()
    cj.wait()

    lane = lax.iota(jnp.int32, L)
    perms = [lane ^ s for s in (1, 2, 4, 8)]

    def lanesum(v):
        for p in perms:
            v = v + v.at[p].get(mode="promise_in_bounds")
        return v

    def group(g, _):
        acc_i = jnp.zeros((L,), jnp.float32)
        acc_j = jnp.zeros((L,), jnp.float32)
        for r in range(L):
            b = g * L + r
            u0 = rows_u[b, pl.ds(0, L)]
            u1 = rows_u[b, pl.ds(L, L)]
            u2 = rows_u[b, pl.ds(2 * L, L)]
            u3 = rows_u[b, pl.ds(3 * L, L)]
            i0 = rows_i[b, pl.ds(0, L)]
            i1 = rows_i[b, pl.ds(L, L)]
            i2 = rows_i[b, pl.ds(2 * L, L)]
            i3 = rows_i[b, pl.ds(3 * L, L)]
            j0 = rows_j[b, pl.ds(0, L)]
            j1 = rows_j[b, pl.ds(L, L)]
            j2 = rows_j[b, pl.ds(2 * L, L)]
            j3 = rows_j[b, pl.ds(3 * L, L)]
            pi = (u0 * i0 + u1 * i1) + (u2 * i2 + u3 * i3)
            pj = (u0 * j0 + u1 * j1) + (u2 * j2 + u3 * j3)
            sel = lane == r
            acc_i = jnp.where(sel, lanesum(pi), acc_i)
            acc_j = jnp.where(sel, lanesum(pj), acc_j)
        out_i_v[pl.ds(g * L, L)] = acc_i
        out_j_v[pl.ds(g * L, L)] = acc_j
        return 0

    lax.fori_loop(0, BPW // L, group, 0)

    pltpu.sync_copy(out_i_v, out_i_hbm.at[pl.ds(base, BPW)])
    pltpu.sync_copy(out_j_v, out_j_hbm.at[pl.ds(base, BPW)])


def kernel(user, item_i, item_j, user_emb_weight, item_emb_weight):
    uw_lin = _sc_uconv(user_emb_weight.T).reshape(_UVP, D)
    iw_lin = _tc_detile(item_emb_weight.T).reshape(_VPAD, D)
    return _bpr_kernel(user, item_i, item_j, uw_lin, iw_lin)


# speedup vs baseline: 2.8949x; 1.2051x over previous
import functools

import jax
import jax.numpy as jnp
from jax import lax
from jax.experimental import pallas as pl
from jax.experimental.pallas import tpu as pltpu
from jax.experimental.pallas import tpu_sc as plsc

B = 16384
D = 64
L = 16
V = 1000000


_TBLK = 32768
_TSH = _TBLK.bit_length() - 1
_TGRID = (V + _TBLK - 1) // _TBLK
_VPAD = _TGRID * _TBLK


def _transpose_body(x_ref, out_ref):
    x = x_ref[...]
    y = jnp.swapaxes(x, 0, 1)
    out_ref[...] = jnp.concatenate(
        [y[: _TBLK // 2], y[_TBLK // 2 :]], axis=1)


_tc_detile = pl.pallas_call(
    _transpose_body,
    grid=(_TGRID,),
    in_specs=[pl.BlockSpec((D, _TBLK), lambda b: (0, b))],
    out_specs=pl.BlockSpec((_TBLK // 2, 2 * D), lambda b: (b, 0)),
    out_shape=jax.ShapeDtypeStruct((_VPAD // 2, 2 * D), jnp.float32),
)


_info = plsc.get_sparse_core_info()
NC, NS = _info.num_cores, _info.num_subcores
NW = NC * NS
BPW = B // NW

_mesh = plsc.VectorSubcoreMesh(core_axis_name="c", subcore_axis_name="s")


_NWIN = (V + 127) // 128
_NFULL = _NWIN - 1
_UVP = _NWIN * 128
_WITER = (_NFULL + NW - 1) // NW


def _win_transpose(win, rows, lane, nv):
    @plsc.parallel_loop(0, nv // L, unroll=4)
    def vblock(vb):
        for k in range(L):
            v_idx = (vb * L + k) ^ lane
            for c in range(D // L):
                d_idx = c * L + lane
                g = plsc.load_gather(win, [d_idx, v_idx])
                plsc.store_scatter(rows, [(v_idx << 6) + d_idx], g)


@functools.partial(
    pl.kernel,
    mesh=_mesh,
    out_type=jax.ShapeDtypeStruct((_UVP * D,), jnp.float32),
    compiler_params=pltpu.CompilerParams(needs_layout_passes=False),
    scratch_types=[
        pltpu.VMEM((2 * D, 128), jnp.float32),
        pltpu.VMEM((D, D), jnp.float32),
        pltpu.VMEM((2 * 128 * D,), jnp.float32),
        pltpu.SemaphoreType.DMA,
        pltpu.SemaphoreType.DMA,
    ],
)
def _sc_uconv(uwt_hbm, out_hbm, win2, wintail, rows2, sem_in, sem_out):
    wid = lax.axis_index("s") * NC + lax.axis_index("c")
    lane = lax.iota(jnp.int32, L)

    def fire(i, buf):
        w = i * NW + wid

        @pl.when(w < _NFULL)
        def _():
            pltpu.make_async_copy(
                uwt_hbm.at[:, pl.ds(pl.multiple_of(w * 128, 128), 128)],
                win2.at[pl.ds(buf * D, D), :], sem_in).start()

    def process(i, buf):
        w = i * NW + wid

        @pl.when(w < _NFULL)
        def _():
            @pl.when(i >= 2)
            def _():
                pltpu.make_async_copy(
                    rows2.at[pl.ds(buf * 8192, 8192)], out_hbm.at[pl.ds(0, 8192)],
                    sem_out).wait()

            pltpu.make_async_copy(
                uwt_hbm.at[:, pl.ds(0, 128)], win2.at[pl.ds(buf * D, D), :], sem_in).wait()
            _win_transpose(win2.at[pl.ds(buf * D, D), :], rows2.at[pl.ds(buf * 8192, 8192)], lane, 128)
            pltpu.make_async_copy(
                rows2.at[pl.ds(buf * 8192, 8192)],
                out_hbm.at[pl.ds(pl.multiple_of(w * 8192, 8), 8192)],
                sem_out).start()

    fire(0, 0)

    def pair(i2, _):
        i = i2 * 2
        fire(i + 1, 1)
        process(i, 0)
        fire(i + 2, 0)
        process(i + 1, 1)
        return 0

    lax.fori_loop(0, (_WITER + 1) // 2, pair, 0)

    pltpu.make_async_copy(rows2.at[pl.ds(0, 8192)], out_hbm.at[pl.ds(0, 8192)],
                          sem_out).wait()
    pltpu.make_async_copy(rows2.at[pl.ds(8192, 8192)], out_hbm.at[pl.ds(0, 8192)],
                          sem_out).wait()

    @pl.when(wid == 0)
    def _():
        pltpu.sync_copy(uwt_hbm.at[:, pl.ds(_NFULL * 128, 64)], wintail)

        def drow(d, _):
            for c in range(4):
                g = wintail[d, pl.ds(c * L, L)]
                plsc.store_scatter(
                    rows2.at[pl.ds(0, 8192)], [(c * L + lane) * D + d], g)
            return 0

        lax.fori_loop(0, D, drow, 0)
        pltpu.sync_copy(rows2.at[pl.ds(0, 8192)],
                        out_hbm.at[pl.ds(_NFULL * 8192, 8192)])


@functools.partial(
    pl.kernel,
    mesh=_mesh,
    out_type=(
        jax.ShapeDtypeStruct((B,), jnp.float32),
        jax.ShapeDtypeStruct((B,), jnp.float32),
    ),
    compiler_params=pltpu.CompilerParams(use_tc_tiling_on_sc=False),
    scratch_types=[
        pltpu.VMEM((BPW,), jnp.int32),
        pltpu.VMEM((BPW,), jnp.int32),
        pltpu.VMEM((BPW,), jnp.int32),
        pltpu.VMEM((BPW,), jnp.int32),
        pltpu.VMEM((BPW,), jnp.int32),
        pltpu.VMEM((BPW, D), jnp.float32),
        pltpu.VMEM((BPW, D), jnp.float32),
        pltpu.VMEM((BPW, D), jnp.float32),
        pltpu.VMEM((BPW,), jnp.float32),
        pltpu.VMEM((BPW,), jnp.float32),
        pltpu.SemaphoreType.DMA,
    ],
)
def _bpr_kernel(user_hbm, item_i_hbm, item_j_hbm, uw_hbm, iw_hbm,
                out_i_hbm, out_j_hbm,
                idx_u, idx_i, idx_j, idxq_i, idxq_j, rows_u, rows_i, rows_j,
                out_i_v, out_j_v, sem):
    wid = lax.axis_index("s") * NC + lax.axis_index("c")
    base = wid * BPW

    pltpu.sync_copy(user_hbm.at[pl.ds(base, BPW)], idx_u)
    pltpu.sync_copy(item_i_hbm.at[pl.ds(base, BPW)], idx_i)
    pltpu.sync_copy(item_j_hbm.at[pl.ds(base, BPW)], idx_j)

    cu = pltpu.make_async_copy(uw_hbm.at[idx_u], rows_u, sem)
    cu.start()

    def remap(g, _):
        s = pl.ds(g * L, L)
        for src, dst in ((idx_i, idxq_i), (idx_j, idxq_j)):
            v = src[s]
            dst[s] = (((v >> _TSH) << _TSH)
                      + ((v & (_TBLK // 2 - 1)) << 1)
                      + ((v >> (_TSH - 1)) & 1))
        return 0

    lax.fori_loop(0, BPW // L, remap, 0)

    ci = pltpu.make_async_copy(iw_hbm.at[idxq_i], rows_i, sem)
    cj = pltpu.make_async_copy(iw_hbm.at[idxq_j], rows_j, sem)
    ci.start()
    cj.start()
    cu.wait()
    ci.wait()
    cj.wait()

    lane = lax.iota(jnp.int32, L)
    perms = [lane ^ s for s in (1, 2, 4, 8)]

    def lanesum(v):
        for p in perms:
            v = v + v.at[p].get(mode="promise_in_bounds")
        return v

    def group(g, _):
        acc_i = jnp.zeros((L,), jnp.float32)
        acc_j = jnp.zeros((L,), jnp.float32)
        for r in range(L):
            b = g * L + r
            u0 = rows_u[b, pl.ds(0, L)]
            u1 = rows_u[b, pl.ds(L, L)]
            u2 = rows_u[b, pl.ds(2 * L, L)]
            u3 = rows_u[b, pl.ds(3 * L, L)]
            i0 = rows_i[b, pl.ds(0, L)]
            i1 = rows_i[b, pl.ds(L, L)]
            i2 = rows_i[b, pl.ds(2 * L, L)]
            i3 = rows_i[b, pl.ds(3 * L, L)]
            j0 = rows_j[b, pl.ds(0, L)]
            j1 = rows_j[b, pl.ds(L, L)]
            j2 = rows_j[b, pl.ds(2 * L, L)]
            j3 = rows_j[b, pl.ds(3 * L, L)]
            pi = (u0 * i0 + u1 * i1) + (u2 * i2 + u3 * i3)
            pj = (u0 * j0 + u1 * j1) + (u2 * j2 + u3 * j3)
            sel = lane == r
            acc_i = jnp.where(sel, lanesum(pi), acc_i)
            acc_j = jnp.where(sel, lanesum(pj), acc_j)
        out_i_v[pl.ds(g * L, L)] = acc_i
        out_j_v[pl.ds(g * L, L)] = acc_j
        return 0

    lax.fori_loop(0, BPW // L, group, 0)

    pltpu.sync_copy(out_i_v, out_i_hbm.at[pl.ds(base, BPW)])
    pltpu.sync_copy(out_j_v, out_j_hbm.at[pl.ds(base, BPW)])


def kernel(user, item_i, item_j, user_emb_weight, item_emb_weight):
    uw_lin = _sc_uconv(user_emb_weight.T).reshape(_UVP, D)
    iw_lin = _tc_detile(item_emb_weight.T).reshape(_VPAD, D)
    return _bpr_kernel(user, item_i, item_j, uw_lin, iw_lin)
